# Initial kernel scaffold; baseline (speedup 1.0000x reference)
import jax
import jax.numpy as jnp
from jax.experimental import pallas as pl


def kernel(dy_positions, dy_feats, box_positions, box_feats, dy_indxs, box_indxs, W_cc1, W_cc2, W_cc3, W_cc4, fc1_w, fc1_b, fc2_w, fc2_b, fc3_w, fc3_b, fc4_w, fc4_b):
    raise NotImplementedError("write your pallas kernel here")


# speedup vs baseline: 22.3475x; 22.3475x over previous
import functools

import jax
import jax.numpy as jnp
from jax import lax
from jax.experimental import pallas as pl
from jax.experimental.pallas import tpu as pltpu
from jax.experimental.pallas import tpu_sc as plsc

N = 50000
M = 10000
K = 16

NC = 2
NS = 16
NW = NC * NS
L = 16

NPW = 1568
NP = NPW * NW
EPW = NPW * K
CP = 8
EC = CP * K
NCH = NPW // CP

BR = 512
MPAD = 10240


_SC_PARAMS = pltpu.CompilerParams(needs_layout_passes=False,
                                  use_tc_tiling_on_sc=False)


def _mesh():
    return plsc.VectorSubcoreMesh(core_axis_name="c", subcore_axis_name="s")


def _wid():
    return lax.axis_index("s") * NC + lax.axis_index("c")


@functools.partial(
    pl.kernel,
    out_type=[
        jax.ShapeDtypeStruct((NP * K,), jnp.int32),
        jax.ShapeDtypeStruct((NP * K,), jnp.float32),
        jax.ShapeDtypeStruct((NP * K,), jnp.int32),
        jax.ShapeDtypeStruct((NP * K,), jnp.float32),
    ],
    mesh=_mesh(),
    scratch_types=[
        pltpu.VMEM((NP,), jnp.float32),
        pltpu.VMEM((NPW,), jnp.float32),
        pltpu.VMEM((EPW,), jnp.int32),
        pltpu.VMEM((EPW,), jnp.float32),
    ],
    compiler_params=_SC_PARAMS,
)
def _preprocess(dyx, dyy, dyz, bxx, bxy, bxz, dyi, bxi,
                dyo, dyw, bxo, bxw, tab, qb, ob, sb):
    wid = _wid()
    ebase = wid * EPW
    pbase = wid * NPW
    qsrcs = (dyx, dyy, dyz)

    def coord_body(c):
        def body(p, carry):
            ev = ob[pl.ds(p * K, K)]
            raw = ev if c == 0 else jax.lax.shift_right_logical(ev, 2)
            xs = plsc.load_gather(tab, [raw])
            qsplat = plsc.load_gather(qb, [jnp.zeros((K,), jnp.int32) + p])
            rel = xs - qsplat
            r2 = rel * rel
            pos = (rel > 0).astype(jnp.int32)
            if c == 0:
                sb[pl.ds(p * K, K)] = r2
                ob[pl.ds(p * K, K)] = ev * 4 + pos * 2
            elif c == 1:
                sb[pl.ds(p * K, K)] = sb[pl.ds(p * K, K)] + r2
                ob[pl.ds(p * K, K)] = ev + pos
            else:
                sb[pl.ds(p * K, K)] = jnp.exp(-(sb[pl.ds(p * K, K)] + r2))
            return carry
        return body

    for tabs, tlen, idx_in, off_out, w_out in (
        ((dyx, dyy, dyz), NP, dyi, dyo, dyw),
        ((bxx, bxy, bxz), M, bxi, bxo, bxw),
    ):
        pltpu.sync_copy(idx_in.at[pl.ds(ebase, EPW)], ob)
        for c in range(3):
            pltpu.sync_copy(tabs[c], tab.at[pl.ds(0, tlen)])
            pltpu.sync_copy(qsrcs[c].at[pl.ds(pbase, NPW)], qb)
            lax.fori_loop(0, NPW, coord_body(c), 0)
        pltpu.sync_copy(ob, off_out.at[pl.ds(ebase, EPW)])
        pltpu.sync_copy(sb, w_out.at[pl.ds(ebase, EPW)])


def _make_gather(C):
    nsub = C // L

    @functools.partial(
        pl.kernel,
        out_type=jax.ShapeDtypeStruct((NP, C), jnp.float32),
        mesh=_mesh(),
        scratch_types=[
            pltpu.VMEM((EPW,), jnp.int32),
            pltpu.VMEM((EPW,), jnp.float32),
            pltpu.VMEM((2, EC, C), jnp.float32),
            pltpu.VMEM((CP, C), jnp.float32),
            pltpu.SemaphoreType.DMA,
            pltpu.SemaphoreType.DMA,
        ],
        compiler_params=_SC_PARAMS,
    )
    def k(ytab, off, w, out, offb, wb, rows, outb, sem0, sem1):
        wid = _wid()
        ebase = wid * EPW
        pbase = wid * NPW
        sems = (sem0, sem1)
        pltpu.sync_copy(off.at[pl.ds(ebase, EPW)], offb)
        pltpu.sync_copy(w.at[pl.ds(ebase, EPW)], wb)

        def issue(ch, b):
            pltpu.async_copy(
                ytab.at[offb.at[pl.ds(ch * EC, EC)]], rows.at[b], sems[b])

        issue(0, 0)
        issue(1, 1)

        def outer(g, carry):
            for b in range(2):
                ch = g * 2 + b
                pltpu.make_async_copy(
                    ytab.at[offb.at[pl.ds(ch * EC, EC)]], rows.at[b],
                    sems[b]).wait()

                def acc_body(p, inner):
                    e0 = ch * EC + p * K
                    wv = wb[pl.ds(e0, K)]
                    for cb in range(nsub):
                        a = None
                        for kk in range(K):
                            t = wv[kk] * rows[b, p * K + kk,
                                              pl.ds(cb * L, L)]
                            a = t if a is None else a + t
                        outb[p, pl.ds(cb * L, L)] = a
                    return inner

                lax.fori_loop(0, CP, acc_body, 0)

                @pl.when(ch + 2 < NCH)
                def _():
                    issue(ch + 2, b)

                pltpu.sync_copy(outb, out.at[pl.ds(pbase + ch * CP, CP)])
            return carry

        lax.fori_loop(0, NCH // 2, outer, 0)

    return k


_gather32 = _make_gather(32)
_gather64 = _make_gather(64)
_gather16 = _make_gather(16)


def _dot(a, b):
    return jax.lax.dot_general(
        a, b, (((1,), (0,)), ((), ())),
        precision=lax.Precision.HIGHEST,
        preferred_element_type=jnp.float32)


def _mm_body(x, w, o):
    o[...] = _dot(x[...], w[...])


def _tc_matmul(x, w):
    R, Cin = x.shape
    Cout = w.shape[1]
    return pl.pallas_call(
        _mm_body,
        grid=(R // BR,),
        in_specs=[pl.BlockSpec((BR, Cin), lambda i: (i, 0)),
                  pl.BlockSpec((Cin, Cout), lambda i: (0, 0))],
        out_specs=pl.BlockSpec((BR, Cout), lambda i: (i, 0)),
        out_shape=jax.ShapeDtypeStruct((R, Cout), jnp.float32),
    )(x, w)


def _l1_body(bcc, dcc, ft, f1w, f1b, w2c, x1o, y2o):
    self1 = _dot(ft[...], f1w[...]) + f1b[...]
    x1 = jnp.maximum(
        jnp.concatenate([bcc[...], dcc[...], self1], axis=1), 0.0)
    x1o[...] = x1
    y2o[...] = _dot(x1, w2c[...])


def _l2_body(cc2, x1, f2w, f2b, w3c, x2o, y3o):
    x2 = jnp.maximum(cc2[...], 0.0) + _dot(x1[...], f2w[...]) + f2b[...]
    x2o[...] = x2
    y3o[...] = _dot(x2, w3c[...])


def _l3_body(cc3, x2, f3w, f3b, w4c, x3o, y4o):
    x3 = _dot(x2[...], f3w[...]) + f3b[...] + cc3[...]
    x3o[...] = x3
    y4o[...] = _dot(x3, w4c[...])


def _l4_body(cc4, x3, f4w, f4b, xo):
    xo[...] = _dot(x3[...], f4w[...]) + f4b[...] + cc4[...]


def _row_spec(c):
    return pl.BlockSpec((BR, c), lambda i: (i, 0))


def _full_spec(r, c):
    return pl.BlockSpec((r, c), lambda i: (0, 0))


def _tc_fused(body, ins, full_shapes, out_cols):
    specs = []
    args = []
    for a, blocked in ins:
        args.append(a)
        if blocked:
            specs.append(_row_spec(a.shape[1]))
        else:
            specs.append(_full_spec(*a.shape))
    outs = [jax.ShapeDtypeStruct((NP, c), jnp.float32) for c in out_cols]
    return pl.pallas_call(
        body,
        grid=(NP // BR,),
        in_specs=specs,
        out_specs=[_row_spec(c) for c in out_cols],
        out_shape=outs,
    )(*args)


def kernel(dy_positions, dy_feats, box_positions, box_feats, dy_indxs,
           box_indxs, W_cc1, W_cc2, W_cc3, W_cc4,
           fc1_w, fc1_b, fc2_w, fc2_b, fc3_w, fc3_b, fc4_w, fc4_b):
    dyp = jnp.pad(dy_positions, ((0, NP - N), (0, 0)))
    dyf = jnp.pad(dy_feats, ((0, NP - N), (0, 6)))
    bxf = jnp.pad(box_feats, ((0, MPAD - M), (0, 6)))
    dyi = jnp.pad(dy_indxs, ((0, NP - N), (0, 0))).reshape(-1)
    bxi = jnp.pad(box_indxs, ((0, NP - N), (0, 0))).reshape(-1)
    dyx, dyy, dyz = dyp[:, 0], dyp[:, 1], dyp[:, 2]
    bxx, bxy, bxz = (box_positions[:, 0], box_positions[:, 1],
                     box_positions[:, 2])

    w1c = jnp.transpose(W_cc1, (1, 0, 2)).reshape(2, 128)
    w1c = jnp.pad(w1c, ((0, 6), (0, 0)))
    w2c = jnp.transpose(W_cc2, (1, 0, 2)).reshape(96, 256)
    w3c = jnp.transpose(W_cc3, (1, 0, 2)).reshape(64, 256)
    w4c = jnp.transpose(jnp.pad(W_cc4, ((0, 0), (0, 0), (0, 13))),
                        (1, 0, 2)).reshape(64, 64)
    f1w = jnp.pad(fc1_w, ((0, 6), (0, 0)))
    f4w = jnp.pad(fc4_w, ((0, 0), (0, 13)))
    f4b = jnp.pad(fc4_b, (0, 13))

    dyo, dyw, bxo, bxw = _preprocess(dyx, dyy, dyz, bxx, bxy, bxz, dyi, bxi)

    y1d = _tc_matmul(dyf, w1c).reshape(NP * 4, 32)
    y1b = _tc_matmul(bxf, w1c).reshape(MPAD * 4, 32)
    dy_cc = _gather32(y1d, dyo, dyw)
    box_cc = _gather32(y1b, bxo, bxw)
    x1, y2 = _tc_fused(
        _l1_body,
        [(box_cc, True), (dy_cc, True), (dyf, True),
         (f1w, False), (fc1_b.reshape(1, 32), False), (w2c, False)],
        None, [96, 256])

    cc2 = _gather64(y2.reshape(NP * 4, 64), dyo, dyw)
    x2, y3 = _tc_fused(
        _l2_body,
        [(cc2, True), (x1, True), (fc2_w, False),
         (fc2_b.reshape(1, 64), False), (w3c, False)],
        None, [64, 256])

    cc3 = _gather64(y3.reshape(NP * 4, 64), dyo, dyw)
    x3, y4 = _tc_fused(
        _l3_body,
        [(cc3, True), (x2, True), (fc3_w, False),
         (fc3_b.reshape(1, 64), False), (w4c, False)],
        None, [64, 64])

    cc4 = _gather16(y4.reshape(NP * 4, 16), dyo, dyw)
    (x4,) = _tc_fused(
        _l4_body,
        [(cc4, True), (x3, True), (f4w, False),
         (f4b.reshape(1, 16), False)],
        None, [16])

    return x4[:N, :3]


# speedup vs baseline: 24.2503x; 1.0851x over previous
import functools

import jax
import jax.numpy as jnp
from jax import lax
from jax.experimental import pallas as pl
from jax.experimental.pallas import tpu as pltpu
from jax.experimental.pallas import tpu_sc as plsc

N = 50000
M = 10000
K = 16

NC = 2
NS = 16
NW = NC * NS
L = 16

NPW = 1568
NP = NPW * NW
EPW = NPW * K
CP = 8
EC = CP * K
NCH = NPW // CP
NBUF = 4

BR = 512
MPAD = 10240

_SC_PARAMS = pltpu.CompilerParams(needs_layout_passes=False,
                                  use_tc_tiling_on_sc=False)


def _mesh():
    return plsc.VectorSubcoreMesh(core_axis_name="c", subcore_axis_name="s")


def _wid():
    return lax.axis_index("s") * NC + lax.axis_index("c")


def _coord_body(c, ob, sb, tab, qb):
    def body(p, carry):
        ev = ob[pl.ds(p * K, K)]
        raw = ev if c == 0 else jax.lax.shift_right_logical(ev, 2)
        xs = plsc.load_gather(tab, [raw])
        qsplat = plsc.load_gather(qb, [jnp.zeros((K,), jnp.int32) + p])
        rel = xs - qsplat
        r2 = rel * rel
        pos = (rel > 0).astype(jnp.int32)
        if c == 0:
            sb[pl.ds(p * K, K)] = r2
            ob[pl.ds(p * K, K)] = ev * 4 + pos * 2
        elif c == 1:
            sb[pl.ds(p * K, K)] = sb[pl.ds(p * K, K)] + r2
            ob[pl.ds(p * K, K)] = ev + pos
        else:
            sb[pl.ds(p * K, K)] = jnp.exp(-(sb[pl.ds(p * K, K)] + r2))
        return carry
    return body


def _edge_phase(tabs, tlen, qsrcs, idx_in, ebase, pbase, ob, sb, tab, qb):
    pltpu.sync_copy(idx_in.at[pl.ds(ebase, EPW)], ob)
    for c in range(3):
        pltpu.sync_copy(tabs[c], tab.at[pl.ds(0, tlen)])
        pltpu.sync_copy(qsrcs[c].at[pl.ds(pbase, NPW)], qb)
        lax.fori_loop(0, NPW, _coord_body(c, ob, sb, tab, qb), 0, unroll=2)


def _gather_sum(ytab, out, offb, wb, rows, outb, gsems, osems, pbase, C):
    nsub = C // L

    def issue(ch, j):
        pltpu.async_copy(
            ytab.at[offb.at[pl.ds(ch * EC, EC)]], rows.at[j], gsems[j])

    for j in range(NBUF):
        issue(j, j)

    def outer(g, carry):
        for j in range(NBUF):
            ch = g * NBUF + j
            oi = j % 2
            pltpu.make_async_copy(
                ytab.at[offb.at[pl.ds(ch * EC, EC)]], rows.at[j],
                gsems[j]).wait()

            @pl.when(ch >= 2)
            def _():
                pltpu.make_async_copy(
                    outb.at[oi], out.at[pl.ds(pbase + (ch - 2) * CP, CP)],
                    osems[oi]).wait()

            def acc_body(p, inner):
                e0 = ch * EC + p * K
                wv = wb[pl.ds(e0, K)]
                for cb in range(nsub):
                    ts = [wv[kk] * rows[j, p * K + kk, pl.ds(cb * L, L)]
                          for kk in range(K)]
                    while len(ts) > 1:
                        ts = [ts[i] + ts[i + 1] for i in range(0, len(ts), 2)]
                    outb[oi, p, pl.ds(cb * L, L)] = ts[0]
                return inner

            lax.fori_loop(0, CP, acc_body, 0)

            @pl.when(ch + NBUF < NCH)
            def _():
                issue(ch + NBUF, j)

            pltpu.async_copy(
                outb.at[oi], out.at[pl.ds(pbase + ch * CP, CP)], osems[oi])
        return carry

    lax.fori_loop(0, NCH // NBUF, outer, 0)
    for ch in (NCH - 2, NCH - 1):
        pltpu.make_async_copy(
            outb.at[ch % 2], out.at[pl.ds(pbase + ch * CP, CP)],
            osems[ch % 2]).wait()


@functools.partial(
    pl.kernel,
    out_type=[
        jax.ShapeDtypeStruct((NP * K,), jnp.int32),
        jax.ShapeDtypeStruct((NP * K,), jnp.float32),
        jax.ShapeDtypeStruct((NP, 32), jnp.float32),
        jax.ShapeDtypeStruct((NP, 32), jnp.float32),
    ],
    mesh=_mesh(),
    scratch_types=[
        pltpu.VMEM((NP,), jnp.float32),
        pltpu.VMEM((NPW,), jnp.float32),
        pltpu.VMEM((EPW,), jnp.int32),
        pltpu.VMEM((EPW,), jnp.float32),
        pltpu.VMEM((NBUF, EC, 32), jnp.float32),
        pltpu.VMEM((2, CP, 32), jnp.float32),
        pltpu.SemaphoreType.DMA,
        pltpu.SemaphoreType.DMA,
        pltpu.SemaphoreType.DMA,
        pltpu.SemaphoreType.DMA,
        pltpu.SemaphoreType.DMA,
        pltpu.SemaphoreType.DMA,
    ],
    compiler_params=_SC_PARAMS,
)
def _stage1(dyx, dyy, dyz, bxx, bxy, bxz, dyi, bxi, y1b, y1d,
            dyo, dyw, bxcc, dycc,
            tab, qb, ob, sb, rows, outb, g0, g1, g2, g3, o0, o1):
    wid = _wid()
    ebase = wid * EPW
    pbase = wid * NPW
    gsems = (g0, g1, g2, g3)
    osems = (o0, o1)
    qsrcs = (dyx, dyy, dyz)

    _edge_phase((bxx, bxy, bxz), M, qsrcs, bxi, ebase, pbase, ob, sb, tab, qb)
    _gather_sum(y1b, bxcc, ob, sb, rows, outb, gsems, osems, pbase, 32)

    _edge_phase(qsrcs, NP, qsrcs, dyi, ebase, pbase, ob, sb, tab, qb)
    pltpu.sync_copy(ob, dyo.at[pl.ds(ebase, EPW)])
    pltpu.sync_copy(sb, dyw.at[pl.ds(ebase, EPW)])
    _gather_sum(y1d, dycc, ob, sb, rows, outb, gsems, osems, pbase, 32)


def _make_gather(C):
    @functools.partial(
        pl.kernel,
        out_type=jax.ShapeDtypeStruct((NP, C), jnp.float32),
        mesh=_mesh(),
        scratch_types=[
            pltpu.VMEM((EPW,), jnp.int32),
            pltpu.VMEM((EPW,), jnp.float32),
            pltpu.VMEM((NBUF, EC, C), jnp.float32),
            pltpu.VMEM((2, CP, C), jnp.float32),
            pltpu.SemaphoreType.DMA,
            pltpu.SemaphoreType.DMA,
            pltpu.SemaphoreType.DMA,
            pltpu.SemaphoreType.DMA,
            pltpu.SemaphoreType.DMA,
            pltpu.SemaphoreType.DMA,
        ],
        compiler_params=_SC_PARAMS,
    )
    def k(ytab, off, w, out, offb, wb, rows, outb, g0, g1, g2, g3, o0, o1):
        wid = _wid()
        ebase = wid * EPW
        pltpu.sync_copy(off.at[pl.ds(ebase, EPW)], offb)
        pltpu.sync_copy(w.at[pl.ds(ebase, EPW)], wb)
        _gather_sum(ytab, out, offb, wb, rows, outb, (g0, g1, g2, g3),
                    (o0, o1), wid * NPW, C)

    return k


_gather64 = _make_gather(64)
_gather16 = _make_gather(16)


def _dot(a, b):
    return jax.lax.dot_general(
        a, b, (((1,), (0,)), ((), ())),
        precision=lax.Precision.HIGHEST,
        preferred_element_type=jnp.float32)


def _mm_body(x, w, o):
    o[...] = _dot(x[...], w[...])


def _tc_matmul(x, w):
    R, Cin = x.shape
    Cout = w.shape[1]
    return pl.pallas_call(
        _mm_body,
        grid=(R // BR,),
        in_specs=[pl.BlockSpec((BR, Cin), lambda i: (i, 0)),
                  pl.BlockSpec((Cin, Cout), lambda i: (0, 0))],
        out_specs=pl.BlockSpec((BR, Cout), lambda i: (i, 0)),
        out_shape=jax.ShapeDtypeStruct((R, Cout), jnp.float32),
    )(x, w)


def _l1_body(bcc, dcc, ft, f1w, f1b, w2c, x1o, y2o):
    self1 = _dot(ft[...], f1w[...]) + f1b[...]
    x1 = jnp.maximum(
        jnp.concatenate([bcc[...], dcc[...], self1], axis=1), 0.0)
    x1o[...] = x1
    y2o[...] = _dot(x1, w2c[...])


def _l2_body(cc2, x1, f2w, f2b, w3c, x2o, y3o):
    x2 = jnp.maximum(cc2[...], 0.0) + _dot(x1[...], f2w[...]) + f2b[...]
    x2o[...] = x2
    y3o[...] = _dot(x2, w3c[...])


def _l3_body(cc3, x2, f3w, f3b, w4c, x3o, y4o):
    x3 = _dot(x2[...], f3w[...]) + f3b[...] + cc3[...]
    x3o[...] = x3
    y4o[...] = _dot(x3, w4c[...])


def _l4_body(cc4, x3, f4w, f4b, xo):
    xo[...] = _dot(x3[...], f4w[...]) + f4b[...] + cc4[...]


def _row_spec(c):
    return pl.BlockSpec((BR, c), lambda i: (i, 0))


def _full_spec(r, c):
    return pl.BlockSpec((r, c), lambda i: (0, 0))


def _tc_fused(body, ins, out_cols):
    specs = []
    args = []
    for a, blocked in ins:
        args.append(a)
        if blocked:
            specs.append(_row_spec(a.shape[1]))
        else:
            specs.append(_full_spec(*a.shape))
    outs = [jax.ShapeDtypeStruct((NP, c), jnp.float32) for c in out_cols]
    return pl.pallas_call(
        body,
        grid=(NP // BR,),
        in_specs=specs,
        out_specs=[_row_spec(c) for c in out_cols],
        out_shape=outs,
    )(*args)


def kernel(dy_positions, dy_feats, box_positions, box_feats, dy_indxs,
           box_indxs, W_cc1, W_cc2, W_cc3, W_cc4,
           fc1_w, fc1_b, fc2_w, fc2_b, fc3_w, fc3_b, fc4_w, fc4_b):
    dyp = jnp.pad(dy_positions, ((0, NP - N), (0, 0)))
    dyf = jnp.pad(dy_feats, ((0, NP - N), (0, 6)))
    bxf = jnp.pad(box_feats, ((0, MPAD - M), (0, 6)))
    dyi = jnp.pad(dy_indxs, ((0, NP - N), (0, 0))).reshape(-1)
    bxi = jnp.pad(box_indxs, ((0, NP - N), (0, 0))).reshape(-1)
    dyx, dyy, dyz = dyp[:, 0], dyp[:, 1], dyp[:, 2]
    bxx, bxy, bxz = (box_positions[:, 0], box_positions[:, 1],
                     box_positions[:, 2])

    w1c = jnp.transpose(W_cc1, (1, 0, 2)).reshape(2, 128)
    w1c = jnp.pad(w1c, ((0, 6), (0, 0)))
    w2c = jnp.transpose(W_cc2, (1, 0, 2)).reshape(96, 256)
    w3c = jnp.transpose(W_cc3, (1, 0, 2)).reshape(64, 256)
    w4c = jnp.transpose(jnp.pad(W_cc4, ((0, 0), (0, 0), (0, 13))),
                        (1, 0, 2)).reshape(64, 64)
    f1w = jnp.pad(fc1_w, ((0, 6), (0, 0)))
    f4w = jnp.pad(fc4_w, ((0, 0), (0, 13)))
    f4b = jnp.pad(fc4_b, (0, 13))

    y1d = _tc_matmul(dyf, w1c).reshape(NP * 4, 32)
    y1b = _tc_matmul(bxf, w1c).reshape(MPAD * 4, 32)
    dyo, dyw, box_cc, dy_cc = _stage1(dyx, dyy, dyz, bxx, bxy, bxz,
                                      dyi, bxi, y1b, y1d)
    x1, y2 = _tc_fused(
        _l1_body,
        [(box_cc, True), (dy_cc, True), (dyf, True),
         (f1w, False), (fc1_b.reshape(1, 32), False), (w2c, False)],
        [96, 256])

    cc2 = _gather64(y2.reshape(NP * 4, 64), dyo, dyw)
    x2, y3 = _tc_fused(
        _l2_body,
        [(cc2, True), (x1, True), (fc2_w, False),
         (fc2_b.reshape(1, 64), False), (w3c, False)],
        [64, 256])

    cc3 = _gather64(y3.reshape(NP * 4, 64), dyo, dyw)
    x3, y4 = _tc_fused(
        _l3_body,
        [(cc3, True), (x2, True), (fc3_w, False),
         (fc3_b.reshape(1, 64), False), (w4c, False)],
        [64, 64])

    cc4 = _gather16(y4.reshape(NP * 4, 16), dyo, dyw)
    (x4,) = _tc_fused(
        _l4_body,
        [(cc4, True), (x3, True), (f4w, False),
         (f4b.reshape(1, 16), False)],
        [16])

    return x4[:N, :3]


# speedup vs baseline: 24.9247x; 1.0278x over previous
import functools

import jax
import jax.numpy as jnp
from jax import lax
from jax.experimental import pallas as pl
from jax.experimental.pallas import tpu as pltpu
from jax.experimental.pallas import tpu_sc as plsc

N = 50000
M = 10000
K = 16

NC = 2
NS = 16
NW = NC * NS
L = 16

NPW = 1568
NP = NPW * NW
EPW = NPW * K
CP = 8
EC = CP * K
NCH = NPW // CP
NBUF = 4

BR = 512
MPAD = 10240

_SC_PARAMS = pltpu.CompilerParams(needs_layout_passes=False,
                                  use_tc_tiling_on_sc=False)


def _mesh():
    return plsc.VectorSubcoreMesh(core_axis_name="c", subcore_axis_name="s")


def _wid():
    return lax.axis_index("s") * NC + lax.axis_index("c")


def _coord_body(c, ob, sb, tab, qb):
    def body(p, carry):
        ev = ob[pl.ds(p * K, K)]
        raw = ev if c == 0 else jax.lax.shift_right_logical(ev, 2)
        xs = plsc.load_gather(tab, [raw])
        qsplat = plsc.load_gather(qb, [jnp.zeros((K,), jnp.int32) + p])
        rel = xs - qsplat
        r2 = rel * rel
        pos = (rel > 0).astype(jnp.int32)
        if c == 0:
            sb[pl.ds(p * K, K)] = r2
            ob[pl.ds(p * K, K)] = ev * 4 + pos * 2
        elif c == 1:
            sb[pl.ds(p * K, K)] = sb[pl.ds(p * K, K)] + r2
            ob[pl.ds(p * K, K)] = ev + pos
        else:
            sb[pl.ds(p * K, K)] = jnp.exp(-(sb[pl.ds(p * K, K)] + r2))
        return carry
    return body


def _edge_phase(tabs, tlen, qsrcs, idx_in, ebase, pbase, ob, sb, tab, qb):
    pltpu.sync_copy(idx_in.at[pl.ds(ebase, EPW)], ob)
    for c in range(3):
        pltpu.sync_copy(tabs[c], tab.at[pl.ds(0, tlen)])
        pltpu.sync_copy(qsrcs[c].at[pl.ds(pbase, NPW)], qb)
        lax.fori_loop(0, NPW, _coord_body(c, ob, sb, tab, qb), 0, unroll=4)


def _gather_sum(ytab, out, offb, wb, rows, outb, gsems, osems, pbase, C,
                out_col=0):
    nsub = C // L

    def odst(ch):
        return out.at[pl.ds(pbase + ch * CP, CP), pl.ds(out_col, C)]

    def issue(ch, j):
        pltpu.async_copy(
            ytab.at[offb.at[pl.ds(ch * EC, EC)]], rows.at[j], gsems[j])

    for j in range(NBUF):
        issue(j, j)

    def outer(g, carry):
        for j in range(NBUF):
            ch = g * NBUF + j
            oi = j % 2
            pltpu.make_async_copy(
                ytab.at[offb.at[pl.ds(ch * EC, EC)]], rows.at[j],
                gsems[j]).wait()

            @pl.when(ch >= 2)
            def _():
                pltpu.make_async_copy(
                    outb.at[oi], odst(ch - 2), osems[oi]).wait()

            def acc_body(p, inner):
                e0 = ch * EC + p * K
                wv = wb[pl.ds(e0, K)]
                for cb in range(nsub):
                    ts = [wv[kk] * rows[j, p * K + kk, pl.ds(cb * L, L)]
                          for kk in range(K)]
                    while len(ts) > 1:
                        ts = [ts[i] + ts[i + 1] for i in range(0, len(ts), 2)]
                    outb[oi, p, pl.ds(cb * L, L)] = ts[0]
                return inner

            lax.fori_loop(0, CP, acc_body, 0)

            @pl.when(ch + NBUF < NCH)
            def _():
                issue(ch + NBUF, j)

            pltpu.async_copy(outb.at[oi], odst(ch), osems[oi])
        return carry

    lax.fori_loop(0, NCH // NBUF, outer, 0)
    for ch in (NCH - 2, NCH - 1):
        pltpu.make_async_copy(
            outb.at[ch % 2], odst(ch), osems[ch % 2]).wait()


@functools.partial(
    pl.kernel,
    out_type=[
        jax.ShapeDtypeStruct((NP * K,), jnp.int32),
        jax.ShapeDtypeStruct((NP * K,), jnp.float32),
        jax.ShapeDtypeStruct((NP, 128), jnp.float32),
    ],
    mesh=_mesh(),
    scratch_types=[
        pltpu.VMEM((NP,), jnp.float32),
        pltpu.VMEM((NPW,), jnp.float32),
        pltpu.VMEM((EPW,), jnp.int32),
        pltpu.VMEM((EPW,), jnp.float32),
        pltpu.VMEM((NBUF, EC, 32), jnp.float32),
        pltpu.VMEM((2, CP, 32), jnp.float32),
        pltpu.SemaphoreType.DMA,
        pltpu.SemaphoreType.DMA,
        pltpu.SemaphoreType.DMA,
        pltpu.SemaphoreType.DMA,
        pltpu.SemaphoreType.DMA,
        pltpu.SemaphoreType.DMA,
    ],
    compiler_params=_SC_PARAMS,
)
def _stage1(dyx, dyy, dyz, bxx, bxy, bxz, dyi, bxi, y1b, y1d,
            dyo, dyw, ccb,
            tab, qb, ob, sb, rows, outb, g0, g1, g2, g3, o0, o1):
    wid = _wid()
    ebase = wid * EPW
    pbase = wid * NPW
    gsems = (g0, g1, g2, g3)
    osems = (o0, o1)
    qsrcs = (dyx, dyy, dyz)

    _edge_phase((bxx, bxy, bxz), M, qsrcs, bxi, ebase, pbase, ob, sb, tab, qb)
    _gather_sum(y1b, ccb, ob, sb, rows, outb, gsems, osems, pbase, 32,
                out_col=0)

    _edge_phase(qsrcs, NP, qsrcs, dyi, ebase, pbase, ob, sb, tab, qb)
    pltpu.sync_copy(ob, dyo.at[pl.ds(ebase, EPW)])
    pltpu.sync_copy(sb, dyw.at[pl.ds(ebase, EPW)])
    _gather_sum(y1d, ccb, ob, sb, rows, outb, gsems, osems, pbase, 32,
                out_col=32)


def _make_gather(C):
    @functools.partial(
        pl.kernel,
        out_type=jax.ShapeDtypeStruct((NP, 128), jnp.float32),
        mesh=_mesh(),
        scratch_types=[
            pltpu.VMEM((EPW,), jnp.int32),
            pltpu.VMEM((EPW,), jnp.float32),
            pltpu.VMEM((NBUF, EC, C), jnp.float32),
            pltpu.VMEM((2, CP, C), jnp.float32),
            pltpu.SemaphoreType.DMA,
            pltpu.SemaphoreType.DMA,
            pltpu.SemaphoreType.DMA,
            pltpu.SemaphoreType.DMA,
            pltpu.SemaphoreType.DMA,
            pltpu.SemaphoreType.DMA,
        ],
        compiler_params=_SC_PARAMS,
    )
    def k(ytab, off, w, out, offb, wb, rows, outb, g0, g1, g2, g3, o0, o1):
        wid = _wid()
        ebase = wid * EPW
        pltpu.sync_copy(off.at[pl.ds(ebase, EPW)], offb)
        pltpu.sync_copy(w.at[pl.ds(ebase, EPW)], wb)
        _gather_sum(ytab, out, offb, wb, rows, outb, (g0, g1, g2, g3),
                    (o0, o1), wid * NPW, C)

    return k


_gather64 = _make_gather(64)
_gather32 = _make_gather(32)


def _dot(a, b):
    return jax.lax.dot_general(
        a, b, (((1,), (0,)), ((), ())),
        precision=lax.Precision.HIGHEST,
        preferred_element_type=jnp.float32)


def _mm_body(x, w, o):
    o[...] = _dot(x[...], w[...])


def _tc_matmul(x, w):
    R, Cin = x.shape
    Cout = w.shape[1]
    return pl.pallas_call(
        _mm_body,
        grid=(R // BR,),
        in_specs=[pl.BlockSpec((BR, Cin), lambda i: (i, 0)),
                  pl.BlockSpec((Cin, Cout), lambda i: (0, 0))],
        out_specs=pl.BlockSpec((BR, Cout), lambda i: (i, 0)),
        out_shape=jax.ShapeDtypeStruct((R, Cout), jnp.float32),
    )(x, w)


def _l1_body(ccb, ft, f1w, f1b, w2c, x1o, y2o):
    self1 = _dot(ft[...], f1w[...]) + f1b[...]
    x1 = jnp.maximum(
        jnp.concatenate([ccb[:, :64], self1], axis=1), 0.0)
    x1o[...] = x1
    y2o[...] = _dot(x1, w2c[...])


def _l2_body(cc2, x1, f2w, f2b, w3c, x2o, y3o):
    x2 = (jnp.maximum(cc2[:, :64], 0.0) + _dot(x1[...], f2w[...])
          + f2b[...])
    x2o[...] = x2
    y3o[...] = _dot(x2, w3c[...])


def _l3_body(cc3, x2, f3w, f3b, w4c, x3o, y4o):
    x3 = _dot(x2[...], f3w[...]) + f3b[...] + cc3[:, :64]
    x3o[...] = x3
    y4o[...] = _dot(x3, w4c[...])


def _l4_body(cc4, x3, f4w, f4b, xo):
    xo[...] = _dot(x3[...], f4w[...]) + f4b[...] + cc4[:, :16]


def _row_spec(c):
    return pl.BlockSpec((BR, c), lambda i: (i, 0))


def _full_spec(r, c):
    return pl.BlockSpec((r, c), lambda i: (0, 0))


def _tc_fused(body, ins, out_cols):
    specs = []
    args = []
    for a, blocked in ins:
        args.append(a)
        if blocked:
            specs.append(_row_spec(a.shape[1]))
        else:
            specs.append(_full_spec(*a.shape))
    outs = [jax.ShapeDtypeStruct((NP, c), jnp.float32) for c in out_cols]
    return pl.pallas_call(
        body,
        grid=(NP // BR,),
        in_specs=specs,
        out_specs=[_row_spec(c) for c in out_cols],
        out_shape=outs,
    )(*args)


def kernel(dy_positions, dy_feats, box_positions, box_feats, dy_indxs,
           box_indxs, W_cc1, W_cc2, W_cc3, W_cc4,
           fc1_w, fc1_b, fc2_w, fc2_b, fc3_w, fc3_b, fc4_w, fc4_b):
    dyp = jnp.pad(dy_positions, ((0, NP - N), (0, 0)))
    dyf = jnp.pad(dy_feats, ((0, NP - N), (0, 6)))
    bxf = jnp.pad(box_feats, ((0, MPAD - M), (0, 6)))
    dyi = jnp.pad(dy_indxs, ((0, NP - N), (0, 0))).reshape(-1)
    bxi = jnp.pad(box_indxs, ((0, NP - N), (0, 0))).reshape(-1)
    dyx, dyy, dyz = dyp[:, 0], dyp[:, 1], dyp[:, 2]
    bxx, bxy, bxz = (box_positions[:, 0], box_positions[:, 1],
                     box_positions[:, 2])

    w1c = jnp.transpose(W_cc1, (1, 0, 2)).reshape(2, 128)
    w1c = jnp.pad(w1c, ((0, 6), (0, 0)))
    w2c = jnp.transpose(W_cc2, (1, 0, 2)).reshape(96, 256)
    w3c = jnp.transpose(W_cc3, (1, 0, 2)).reshape(64, 256)
    w4c = jnp.transpose(jnp.pad(W_cc4, ((0, 0), (0, 0), (0, 29))),
                        (1, 0, 2)).reshape(64, 128)
    f1w = jnp.pad(fc1_w, ((0, 6), (0, 0)))
    f4w = jnp.pad(fc4_w, ((0, 0), (0, 13)))
    f4b = jnp.pad(fc4_b, (0, 13))

    y1d = _tc_matmul(dyf, w1c).reshape(NP * 4, 32)
    y1b = _tc_matmul(bxf, w1c).reshape(MPAD * 4, 32)
    dyo, dyw, ccb = _stage1(dyx, dyy, dyz, bxx, bxy, bxz, dyi, bxi, y1b, y1d)
    x1, y2 = _tc_fused(
        _l1_body,
        [(ccb, True), (dyf, True),
         (f1w, False), (fc1_b.reshape(1, 32), False), (w2c, False)],
        [96, 256])

    cc2 = _gather64(y2.reshape(NP * 4, 64), dyo, dyw)
    x2, y3 = _tc_fused(
        _l2_body,
        [(cc2, True), (x1, True), (fc2_w, False),
         (fc2_b.reshape(1, 64), False), (w3c, False)],
        [64, 256])

    cc3 = _gather64(y3.reshape(NP * 4, 64), dyo, dyw)
    x3, y4 = _tc_fused(
        _l3_body,
        [(cc3, True), (x2, True), (fc3_w, False),
         (fc3_b.reshape(1, 64), False), (w4c, False)],
        [64, 128])

    cc4 = _gather32(y4.reshape(NP * 4, 32), dyo, dyw)
    (x4,) = _tc_fused(
        _l4_body,
        [(cc4, True), (x3, True), (f4w, False),
         (f4b.reshape(1, 16), False)],
        [16])

    return x4[:N, :3]


# speedup vs baseline: 25.4935x; 1.0228x over previous
import functools

import jax
import jax.numpy as jnp
from jax import lax
from jax.experimental import pallas as pl
from jax.experimental.pallas import tpu as pltpu
from jax.experimental.pallas import tpu_sc as plsc

N = 50000
M = 10000
K = 16

NC = 2
NS = 16
NW = NC * NS
L = 16

NPW = 1568
NP = NPW * NW
EPW = NPW * K
CP = 8
EC = CP * K
NCH = NPW // CP
NBUF = 4

BR = 512
MPAD = 10240

_SC_PARAMS = pltpu.CompilerParams(needs_layout_passes=False,
                                  use_tc_tiling_on_sc=False)


def _mesh():
    return plsc.VectorSubcoreMesh(core_axis_name="c", subcore_axis_name="s")


def _wid():
    return lax.axis_index("s") * NC + lax.axis_index("c")


def _coord_body(c, ob, sb, tab, qb):
    def body(p, carry):
        ev = ob[pl.ds(p * K, K)]
        raw = ev if c == 0 else jax.lax.shift_right_logical(ev, 2)
        xs = plsc.load_gather(tab, [raw])
        qsplat = plsc.load_gather(qb, [jnp.zeros((K,), jnp.int32) + p])
        rel = xs - qsplat
        r2 = rel * rel
        pos = (rel > 0).astype(jnp.int32)
        if c == 0:
            sb[pl.ds(p * K, K)] = r2
            ob[pl.ds(p * K, K)] = ev * 4 + pos * 2
        elif c == 1:
            sb[pl.ds(p * K, K)] = sb[pl.ds(p * K, K)] + r2
            ob[pl.ds(p * K, K)] = ev + pos
        else:
            sb[pl.ds(p * K, K)] = jnp.exp(-(sb[pl.ds(p * K, K)] + r2))
        return carry
    return body


def _edge_phase(tabs, tlen, qsrcs, idx_in, ebase, pbase, ob, sb, tab, qb):
    pltpu.sync_copy(idx_in.at[pl.ds(ebase, EPW)], ob)
    for c in range(3):
        pltpu.sync_copy(tabs[c], tab.at[pl.ds(0, tlen)])
        pltpu.sync_copy(qsrcs[c].at[pl.ds(pbase, NPW)],
                        qb.at[pl.ds(0, NPW)])
        lax.fori_loop(0, NPW, _coord_body(c, ob, sb, tab, qb), 0, unroll=4)


BOXSTRIDE = 10240


def _box_phase(btabs, qsrcs, idx_in, ebase, pbase, ob, sb, tab, qb):
    pltpu.sync_copy(idx_in.at[pl.ds(ebase, EPW)], ob)
    for t in range(3):
        pltpu.sync_copy(btabs[t], tab.at[pl.ds(t * BOXSTRIDE, M)])
        pltpu.sync_copy(qsrcs[t].at[pl.ds(pbase, NPW)],
                        qb.at[pl.ds(t * NPW, NPW)])

    def body(p, carry):
        ev = ob[pl.ds(p * K, K)]
        iq = jnp.zeros((K,), jnp.int32) + p
        s = None
        bb = None
        for t in range(3):
            xs = plsc.load_gather(tab.at[pl.ds(t * BOXSTRIDE, M)], [ev])
            q = plsc.load_gather(qb.at[pl.ds(t * NPW, NPW)], [iq])
            rel = xs - q
            r2 = rel * rel
            s = r2 if t == 0 else s + r2
            if t == 0:
                bb = (rel > 0).astype(jnp.int32) * 2
            elif t == 1:
                bb = bb + (rel > 0).astype(jnp.int32)
        sb[pl.ds(p * K, K)] = jnp.exp(-s)
        ob[pl.ds(p * K, K)] = ev * 4 + bb
        return carry

    lax.fori_loop(0, NPW, body, 0, unroll=4)


def _gather_sum(ytab, out, offb, wb, rows, outb, gsems, osems, pbase, C,
                nbuf, out_col=0):
    nsub = C // L

    def odst(ch):
        return out.at[pl.ds(pbase + ch * CP, CP), pl.ds(out_col, C)]

    def issue(ch, j):
        pltpu.async_copy(
            ytab.at[offb.at[pl.ds(ch * EC, EC)]], rows.at[j], gsems[j])

    for j in range(nbuf):
        issue(j, j)

    def outer(g, carry):
        for j in range(nbuf):
            ch = g * nbuf + j
            pltpu.make_async_copy(
                ytab.at[offb.at[pl.ds(ch * EC, EC)]], rows.at[j],
                gsems[j]).wait()

            @pl.when(ch >= nbuf)
            def _():
                pltpu.make_async_copy(
                    outb.at[j], odst(ch - nbuf), osems[j]).wait()

            def acc_body(p, inner):
                e0 = ch * EC + p * K
                wv = wb[pl.ds(e0, K)]
                for cb in range(nsub):
                    ts = [wv[kk] * rows[j, p * K + kk, pl.ds(cb * L, L)]
                          for kk in range(K)]
                    while len(ts) > 1:
                        ts = [ts[i] + ts[i + 1] for i in range(0, len(ts), 2)]
                    outb[j, p, pl.ds(cb * L, L)] = ts[0]
                return inner

            lax.fori_loop(0, CP, acc_body, 0)

            @pl.when(ch + nbuf < NCH)
            def _():
                issue(ch + nbuf, j)

            pltpu.async_copy(outb.at[j], odst(ch), osems[j])
        return carry

    lax.fori_loop(0, NCH // nbuf, outer, 0)
    for ch in range(NCH - nbuf, NCH):
        pltpu.make_async_copy(
            outb.at[ch % nbuf], odst(ch), osems[ch % nbuf]).wait()


@functools.partial(
    pl.kernel,
    out_type=[
        jax.ShapeDtypeStruct((NP * K,), jnp.int32),
        jax.ShapeDtypeStruct((NP * K,), jnp.float32),
        jax.ShapeDtypeStruct((NP, 128), jnp.float32),
    ],
    mesh=_mesh(),
    scratch_types=[
        pltpu.VMEM((NP,), jnp.float32),
        pltpu.VMEM((3 * NPW,), jnp.float32),
        pltpu.VMEM((EPW,), jnp.int32),
        pltpu.VMEM((EPW,), jnp.float32),
        pltpu.VMEM((4, EC, 32), jnp.float32),
        pltpu.VMEM((4, CP, 32), jnp.float32),
    ] + [pltpu.SemaphoreType.DMA] * 8,
    compiler_params=_SC_PARAMS,
)
def _stage1(dyx, dyy, dyz, bxx, bxy, bxz, dyi, bxi, y1b, y1d,
            dyo, dyw, ccb,
            tab, qb, ob, sb, rows, outb,
            g0, g1, g2, g3, o0, o1, o2, o3):
    wid = _wid()
    ebase = wid * EPW
    pbase = wid * NPW
    gsems = (g0, g1, g2, g3)
    osems = (o0, o1, o2, o3)
    qsrcs = (dyx, dyy, dyz)

    _box_phase((bxx, bxy, bxz), qsrcs, bxi, ebase, pbase, ob, sb, tab, qb)
    _gather_sum(y1b, ccb, ob, sb, rows, outb, gsems, osems, pbase, 32,
                nbuf=4, out_col=0)

    _edge_phase(qsrcs, NP, qsrcs, dyi, ebase, pbase, ob, sb, tab, qb)
    pltpu.sync_copy(ob, dyo.at[pl.ds(ebase, EPW)])
    pltpu.sync_copy(sb, dyw.at[pl.ds(ebase, EPW)])
    _gather_sum(y1d, ccb, ob, sb, rows, outb, gsems, osems, pbase, 32,
                nbuf=4, out_col=32)


GNBUF = 7


def _make_gather(C):
    @functools.partial(
        pl.kernel,
        out_type=jax.ShapeDtypeStruct((NP, 128), jnp.float32),
        mesh=_mesh(),
        scratch_types=[
            pltpu.VMEM((EPW,), jnp.int32),
            pltpu.VMEM((EPW,), jnp.float32),
            pltpu.VMEM((GNBUF, EC, C), jnp.float32),
            pltpu.VMEM((GNBUF, CP, C), jnp.float32),
        ] + [pltpu.SemaphoreType.DMA] * (2 * GNBUF),
        compiler_params=_SC_PARAMS,
    )
    def k(ytab, off, w, out, offb, wb, rows, outb,
          g0, g1, g2, g3, g4, g5, g6, o0, o1, o2, o3, o4, o5, o6):
        wid = _wid()
        ebase = wid * EPW
        pltpu.sync_copy(off.at[pl.ds(ebase, EPW)], offb)
        pltpu.sync_copy(w.at[pl.ds(ebase, EPW)], wb)
        _gather_sum(ytab, out, offb, wb, rows, outb,
                    (g0, g1, g2, g3, g4, g5, g6),
                    (o0, o1, o2, o3, o4, o5, o6), wid * NPW, C, nbuf=GNBUF)

    return k


_gather64 = _make_gather(64)
_gather32 = _make_gather(32)


def _dot(a, b):
    return jax.lax.dot_general(
        a, b, (((1,), (0,)), ((), ())),
        precision=lax.Precision.HIGHEST,
        preferred_element_type=jnp.float32)


def _mm_body(x, w, o):
    o[...] = _dot(x[...], w[...])


def _tc_matmul(x, w):
    R, Cin = x.shape
    Cout = w.shape[1]
    return pl.pallas_call(
        _mm_body,
        grid=(R // BR,),
        in_specs=[pl.BlockSpec((BR, Cin), lambda i: (i, 0)),
                  pl.BlockSpec((Cin, Cout), lambda i: (0, 0))],
        out_specs=pl.BlockSpec((BR, Cout), lambda i: (i, 0)),
        out_shape=jax.ShapeDtypeStruct((R, Cout), jnp.float32),
    )(x, w)


def _l1_body(ccb, ft, f1w, f1b, w2c, x1o, y2o):
    self1 = _dot(ft[...], f1w[...]) + f1b[...]
    x1 = jnp.maximum(
        jnp.concatenate([ccb[:, :64], self1], axis=1), 0.0)
    x1o[...] = x1
    y2o[...] = _dot(x1, w2c[...])


def _l2_body(cc2, x1, f2w, f2b, w3c, x2o, y3o):
    x2 = (jnp.maximum(cc2[:, :64], 0.0) + _dot(x1[...], f2w[...])
          + f2b[...])
    x2o[...] = x2
    y3o[...] = _dot(x2, w3c[...])


def _l3_body(cc3, x2, f3w, f3b, w4c, x3o, y4o):
    x3 = _dot(x2[...], f3w[...]) + f3b[...] + cc3[:, :64]
    x3o[...] = x3
    y4o[...] = _dot(x3, w4c[...])


def _l4_body(cc4, x3, f4w, f4b, xo):
    xo[...] = _dot(x3[...], f4w[...]) + f4b[...] + cc4[:, :16]


def _row_spec(c):
    return pl.BlockSpec((BR, c), lambda i: (i, 0))


def _full_spec(r, c):
    return pl.BlockSpec((r, c), lambda i: (0, 0))


def _tc_fused(body, ins, out_cols):
    specs = []
    args = []
    for a, blocked in ins:
        args.append(a)
        if blocked:
            specs.append(_row_spec(a.shape[1]))
        else:
            specs.append(_full_spec(*a.shape))
    outs = [jax.ShapeDtypeStruct((NP, c), jnp.float32) for c in out_cols]
    return pl.pallas_call(
        body,
        grid=(NP // BR,),
        in_specs=specs,
        out_specs=[_row_spec(c) for c in out_cols],
        out_shape=outs,
    )(*args)


def kernel(dy_positions, dy_feats, box_positions, box_feats, dy_indxs,
           box_indxs, W_cc1, W_cc2, W_cc3, W_cc4,
           fc1_w, fc1_b, fc2_w, fc2_b, fc3_w, fc3_b, fc4_w, fc4_b):
    dyp = jnp.pad(dy_positions, ((0, NP - N), (0, 0)))
    dyf = jnp.pad(dy_feats, ((0, NP - N), (0, 6)))
    bxf = jnp.pad(box_feats, ((0, MPAD - M), (0, 6)))
    dyi = jnp.pad(dy_indxs, ((0, NP - N), (0, 0))).reshape(-1)
    bxi = jnp.pad(box_indxs, ((0, NP - N), (0, 0))).reshape(-1)
    dyx, dyy, dyz = dyp[:, 0], dyp[:, 1], dyp[:, 2]
    bxx, bxy, bxz = (box_positions[:, 0], box_positions[:, 1],
                     box_positions[:, 2])

    w1c = jnp.transpose(W_cc1, (1, 0, 2)).reshape(2, 128)
    w1c = jnp.pad(w1c, ((0, 6), (0, 0)))
    w2c = jnp.transpose(W_cc2, (1, 0, 2)).reshape(96, 256)
    w3c = jnp.transpose(W_cc3, (1, 0, 2)).reshape(64, 256)
    w4c = jnp.transpose(jnp.pad(W_cc4, ((0, 0), (0, 0), (0, 29))),
                        (1, 0, 2)).reshape(64, 128)
    f1w = jnp.pad(fc1_w, ((0, 6), (0, 0)))
    f4w = jnp.pad(fc4_w, ((0, 0), (0, 13)))
    f4b = jnp.pad(fc4_b, (0, 13))

    y1d = _tc_matmul(dyf, w1c).reshape(NP * 4, 32)
    y1b = _tc_matmul(bxf, w1c).reshape(MPAD * 4, 32)
    dyo, dyw, ccb = _stage1(dyx, dyy, dyz, bxx, bxy, bxz, dyi, bxi, y1b, y1d)
    x1, y2 = _tc_fused(
        _l1_body,
        [(ccb, True), (dyf, True),
         (f1w, False), (fc1_b.reshape(1, 32), False), (w2c, False)],
        [96, 256])

    cc2 = _gather64(y2.reshape(NP * 4, 64), dyo, dyw)
    x2, y3 = _tc_fused(
        _l2_body,
        [(cc2, True), (x1, True), (fc2_w, False),
         (fc2_b.reshape(1, 64), False), (w3c, False)],
        [64, 256])

    cc3 = _gather64(y3.reshape(NP * 4, 64), dyo, dyw)
    x3, y4 = _tc_fused(
        _l3_body,
        [(cc3, True), (x2, True), (fc3_w, False),
         (fc3_b.reshape(1, 64), False), (w4c, False)],
        [64, 128])

    cc4 = _gather32(y4.reshape(NP * 4, 32), dyo, dyw)
    (x4,) = _tc_fused(
        _l4_body,
        [(cc4, True), (x3, True), (f4w, False),
         (f4b.reshape(1, 16), False)],
        [16])

    return x4[:N, :3]
